# Initial kernel scaffold; baseline (speedup 1.0000x reference)
import jax
import jax.numpy as jnp
from jax.experimental import pallas as pl


def kernel(emb1, n_src, ns_tgt, W, a1, a2):
    raise NotImplementedError("write your pallas kernel here")


# speedup vs baseline: 199.9202x; 199.9202x over previous
import jax
import jax.numpy as jnp
from jax.experimental import pallas as pl
from jax.experimental.pallas import tpu as pltpu

_N_HEAD = 16


def _proj_kernel(w_ref, ac_ref, emb_ref, a12_ref):
    h = jax.lax.dot_general(emb_ref[0].astype(jnp.bfloat16), w_ref[...],
                            (((1,), (1,)), ((), ())),
                            preferred_element_type=jnp.float32)
    a12_ref[0] = jax.lax.dot_general(h.astype(jnp.bfloat16), ac_ref[...],
                                     (((1,), (0,)), ((), ())),
                                     preferred_element_type=jnp.float32)


def _rank_kernel(nsrc_ref, nstgt_ref, a2r_ref, a2c_ref, cmask_ref):
    b = pl.program_id(0)
    nt = nstgt_ref[b]
    kk = (2 * nsrc_ref[b]) // 5
    a2_row = a2c_ref[0]
    n = a2_row.shape[-1]
    colids = jax.lax.broadcasted_iota(jnp.int32, (1, n), 1)
    acc = jnp.zeros((1, n), jnp.int32)
    ch = 256
    for s in range(0, n, ch):
        col = a2r_ref[0, s:s + ch, :]
        rid = jax.lax.broadcasted_iota(jnp.int32, (ch, 1), 0) + s
        validr = rid < nt
        beats = (col > a2_row) | ((col == a2_row) & (rid < colids))
        acc = acc + jnp.sum((beats & validr).astype(jnp.int32),
                            axis=0, keepdims=True)
    cm = (colids < nt) & (acc < kk)
    cmask_ref[0] = cm.astype(jnp.float32)


def _out_kernel(nsrc_ref, a1r_ref, a2r_ref, cmr_ref, a1c_ref, a2c_ref,
                cmc_ref, out_ref):
    b = pl.program_id(0)
    ti = pl.program_id(1)
    n = nsrc_ref[b]
    scale = jnp.float32(5.0) / (2 * n).astype(jnp.float32)
    ai = a1r_ref[0]
    a2i = a2r_ref[0]
    cmi = cmr_ref[0]
    aj = a1c_ref[0]
    a2j = a2c_ref[0]
    cmj = cmc_ref[0]
    tm = ai.shape[0]
    tn = aj.shape[-1]
    rid = ti * tm + jax.lax.broadcasted_iota(jnp.int32, (tm, 1), 0)
    cid = jax.lax.broadcasted_iota(jnp.int32, (1, tn), 1)
    v = jnp.maximum((ai + a2j) * jnp.float32(1.0 / _N_HEAD), 0.0)
    v = jnp.where((rid < n) & (cmj > 0), v, 0.0)
    gate = ((aj + a2i) > 0) & (cid < n) & (cmi > 0)
    out_ref[0] = scale * jnp.tanh(jnp.where(gate, v, 0.0))


def kernel(emb1, n_src, ns_tgt, W, a1, a2):
    B, N, IN_F = emb1.shape
    OUT_F = W.shape[0]
    ac = jnp.concatenate([a1, a2], axis=1).astype(jnp.bfloat16)
    wb = W.astype(jnp.bfloat16)

    BM = 512
    a12 = pl.pallas_call(
        _proj_kernel,
        grid=(B, N // BM),
        in_specs=[
            pl.BlockSpec((OUT_F, IN_F), lambda b, i: (0, 0)),
            pl.BlockSpec((OUT_F, 2), lambda b, i: (0, 0)),
            pl.BlockSpec((1, BM, IN_F), lambda b, i: (b, i, 0)),
        ],
        out_specs=pl.BlockSpec((1, BM, 2), lambda b, i: (b, i, 0)),
        out_shape=jax.ShapeDtypeStruct((B, N, 2), jnp.float32),
    )(wb, ac, emb1)

    a12_c = jnp.transpose(a12, (0, 2, 1))
    a1r = a12[:, :, 0:1]
    a2r = a12[:, :, 1:2]
    a1c = a12_c[:, 0:1, :]
    a2c = a12_c[:, 1:2, :]

    cmc = pl.pallas_call(
        _rank_kernel,
        grid=(B,),
        in_specs=[
            pl.BlockSpec(memory_space=pltpu.SMEM),
            pl.BlockSpec(memory_space=pltpu.SMEM),
            pl.BlockSpec((1, N, 1), lambda b: (b, 0, 0)),
            pl.BlockSpec((1, 1, N), lambda b: (b, 0, 0)),
        ],
        out_specs=pl.BlockSpec((1, 1, N), lambda b: (b, 0, 0)),
        out_shape=jax.ShapeDtypeStruct((B, 1, N), jnp.float32),
    )(n_src, ns_tgt, a2r, a2c)
    cmr = jnp.transpose(cmc, (0, 2, 1))

    TM = 256
    vec_c = pl.BlockSpec((1, 1, N), lambda b, i: (b, 0, 0))
    vec_r = pl.BlockSpec((1, TM, 1), lambda b, i: (b, i, 0))
    out = pl.pallas_call(
        _out_kernel,
        grid=(B, N // TM),
        in_specs=[
            pl.BlockSpec(memory_space=pltpu.SMEM),
            vec_r, vec_r, vec_r, vec_c, vec_c, vec_c,
        ],
        out_specs=pl.BlockSpec((1, TM, N), lambda b, i: (b, i, 0)),
        out_shape=jax.ShapeDtypeStruct((B, N, N), jnp.float32),
    )(n_src, a1r, a2r, cmr, a1c, a2c, cmc)
    return out
